# baseline (speedup 1.0000x reference)
import functools

import jax
import jax.numpy as jnp
from jax import lax
from jax.experimental import pallas as pl
from jax.experimental.pallas import tpu as pltpu
from jax.experimental.pallas import tpu_sc as plsc

N = 10000
D = 128
E = 320000

NC = 2
NS = 16
NW = NC * NS
LANES = 16

EPW = E // NW
K = 80
NCH = EPW // K
RPT = 624
ZR = 208

_mesh = plsc.VectorSubcoreMesh(core_axis_name="c", subcore_axis_name="s")


NBUF = 5
NIT = NCH // NBUF


@functools.partial(
    pl.kernel,
    mesh=_mesh,
    out_type=jax.ShapeDtypeStruct((NW * N,), jnp.float32),
    compiler_params=pltpu.CompilerParams(needs_layout_passes=False),
    scratch_types=[
        pltpu.VMEM((EPW,), jnp.int32),
        pltpu.VMEM((N,), jnp.float32),
    ],
)
def _sc_degree(dst_hbm, out_hbm, didx, cnt):
    c = lax.axis_index("c")
    s = lax.axis_index("s")
    wid = s * NC + c

    pltpu.sync_copy(dst_hbm.at[pl.ds(wid * EPW, EPW)], didx)

    zero16 = jnp.zeros((LANES,), jnp.float32)
    one16 = jnp.ones((LANES,), jnp.float32)

    def zfill(i, _):
        cnt[pl.ds(i * LANES, LANES)] = zero16
        return 0
    lax.fori_loop(0, N // LANES, zfill, 0)

    def count(j, _):
        idx = didx[pl.ds(j * LANES, LANES)]
        plsc.addupdate_scatter(cnt, [idx], one16)
        return 0
    lax.fori_loop(0, EPW // LANES, count, 0)

    pltpu.sync_copy(cnt, out_hbm.at[pl.ds(wid * N, N)])


KA = 80
NBA = 3
NCHA = EPW // KA
NITA = NCHA // NBA
TAIL = NCHA - NITA * NBA


@functools.partial(
    pl.kernel,
    mesh=_mesh,
    out_type=jax.ShapeDtypeStruct((NC * N, D), jnp.float32),
    scratch_types=[
        [pltpu.VMEM((KA,), jnp.int32)] * NBA,
        [pltpu.VMEM((KA,), jnp.int32)] * NBA,
        [pltpu.VMEM((KA, D), jnp.float32)] * NBA,
        pltpu.VMEM_SHARED((N, D), jnp.float32),
        [pltpu.SemaphoreType.DMA] * NBA,
        [pltpu.SemaphoreType.DMA] * NBA,
        [pltpu.SemaphoreType.DMA] * NBA,
    ],
)
def _sc_aggregate(xs_hbm, src_hbm, dst_hbm, zeros_hbm, out_hbm,
                  sidx, didx, rows, acc, isem, gsem, ssem):
    c = lax.axis_index("c")
    s = lax.axis_index("s")
    wid = s * NC + c

    rbase = s * RPT
    pltpu.sync_copy(zeros_hbm.at[pl.ds(0, RPT)],
                    acc.at[pl.ds(rbase, RPT)])

    @pl.when(s == NS - 1)
    def _ztail():
        pltpu.sync_copy(zeros_hbm.at[pl.ds(0, 16)], acc.at[pl.ds(N - 16, 16)])

    plsc.subcore_barrier()

    ebase = wid * EPW

    def icopy(i, b):
        eb = ebase + i * KA
        pltpu.async_copy(src_hbm.at[pl.ds(eb, KA)], sidx[b], isem[b])
        pltpu.async_copy(dst_hbm.at[pl.ds(eb, KA)], didx[b], isem[b])

    def iwait(b):
        pltpu.make_async_copy(src_hbm.at[pl.ds(0, KA)], sidx[b],
                              isem[b]).wait()
        pltpu.make_async_copy(dst_hbm.at[pl.ds(0, KA)], didx[b],
                              isem[b]).wait()

    def gather(b):
        pltpu.async_copy(xs_hbm.at[sidx[b]], rows[b], gsem[b])

    def gwait(b):
        pltpu.make_async_copy(xs_hbm.at[sidx[b]], rows[b], gsem[b]).wait()

    def scatter(b):
        pltpu.async_copy(rows[b], acc.at[didx[b]], ssem[b], add=True)

    def swait(b):
        pltpu.make_async_copy(rows[b], acc.at[didx[b]], ssem[b]).wait()

    for b in range(NBA):
        icopy(b, b)

    def step(g, _):
        for b in range(NBA):
            iwait(b)
            gather(b)
        for b in range(NBA):
            gwait(b)
            scatter(b)

        @pl.when(g < NITA - 1)
        def _prefetch():
            for b in range(NBA):
                swait(b)
                icopy(g * NBA + NBA + b, b)
        return 0
    lax.fori_loop(0, NITA, step, 0)

    for b in range(TAIL):
        swait(b)
        icopy(NITA * NBA + b, b)
    for b in range(TAIL):
        iwait(b)
        gather(b)
    for b in range(TAIL):
        gwait(b)
        scatter(b)
    for b in range(NBA):
        swait(b)

    plsc.subcore_barrier()
    pltpu.sync_copy(acc.at[pl.ds(rbase, RPT)],
                    out_hbm.at[pl.ds(c * N + rbase, RPT)])

    @pl.when(s == NS - 1)
    def _otail():
        pltpu.sync_copy(acc.at[pl.ds(N - 16, 16)],
                        out_hbm.at[pl.ds(c * N + N - 16, 16)])


_RB = 1000
_HIGH = lax.Precision.HIGHEST


def _prep_body(x_ref, w_ref, deg_ref, xs_ref, self_ref, dis_ref, inv_ref):
    deg = deg_ref[...]
    dis = lax.rsqrt(deg)
    inv = 1.0 / deg
    xw = jnp.dot(x_ref[...], w_ref[...], precision=_HIGH)
    xs_ref[...] = xw * dis
    self_ref[...] = xw * inv
    dis_ref[...] = dis
    inv_ref[...] = inv


def _tc_prep(x, w1, deg):
    grid = (N // _RB,)
    return pl.pallas_call(
        _prep_body,
        grid=grid,
        in_specs=[
            pl.BlockSpec((_RB, D), lambda i: (i, 0)),
            pl.BlockSpec((D, D), lambda i: (0, 0)),
            pl.BlockSpec((_RB, 1), lambda i: (i, 0)),
        ],
        out_specs=[
            pl.BlockSpec((_RB, D), lambda i: (i, 0)),
            pl.BlockSpec((_RB, D), lambda i: (i, 0)),
            pl.BlockSpec((_RB, 1), lambda i: (i, 0)),
            pl.BlockSpec((_RB, 1), lambda i: (i, 0)),
        ],
        out_shape=[
            jax.ShapeDtypeStruct((N, D), jnp.float32),
            jax.ShapeDtypeStruct((N, D), jnp.float32),
            jax.ShapeDtypeStruct((N, 1), jnp.float32),
            jax.ShapeDtypeStruct((N, 1), jnp.float32),
        ],
    )(x, w1, deg)


def _mid_body(q0_ref, q1_ref, self_ref, b_ref, w_ref, dis_ref, inv_ref,
              xs_ref, self2_ref):
    dis = dis_ref[...]
    inv = inv_ref[...]
    h = dis * (q0_ref[...] + q1_ref[...]) + self_ref[...] + b_ref[...]
    h = jnp.maximum(h, 0.0)
    xw = jnp.dot(h, w_ref[...], precision=_HIGH)
    xs_ref[...] = xw * dis
    self2_ref[...] = xw * inv


def _tc_mid(q0, q1, self1, b1, w2, dis, inv):
    grid = (N // _RB,)
    return pl.pallas_call(
        _mid_body,
        grid=grid,
        in_specs=[
            pl.BlockSpec((_RB, D), lambda i: (i, 0)),
            pl.BlockSpec((_RB, D), lambda i: (i, 0)),
            pl.BlockSpec((_RB, D), lambda i: (i, 0)),
            pl.BlockSpec((1, D), lambda i: (0, 0)),
            pl.BlockSpec((D, D), lambda i: (0, 0)),
            pl.BlockSpec((_RB, 1), lambda i: (i, 0)),
            pl.BlockSpec((_RB, 1), lambda i: (i, 0)),
        ],
        out_specs=[
            pl.BlockSpec((_RB, D), lambda i: (i, 0)),
            pl.BlockSpec((_RB, D), lambda i: (i, 0)),
        ],
        out_shape=[
            jax.ShapeDtypeStruct((N, D), jnp.float32),
            jax.ShapeDtypeStruct((N, D), jnp.float32),
        ],
    )(q0, q1, self1, b1, w2, dis, inv)


def _final_body(q0_ref, q1_ref, self_ref, b_ref, dis_ref, out_ref):
    out_ref[...] = (dis_ref[...] * (q0_ref[...] + q1_ref[...])
                    + self_ref[...] + b_ref[...])


def _tc_final(q0, q1, self2, b2, dis):
    grid = (N // _RB,)
    return pl.pallas_call(
        _final_body,
        grid=grid,
        in_specs=[
            pl.BlockSpec((_RB, D), lambda i: (i, 0)),
            pl.BlockSpec((_RB, D), lambda i: (i, 0)),
            pl.BlockSpec((_RB, D), lambda i: (i, 0)),
            pl.BlockSpec((1, D), lambda i: (0, 0)),
            pl.BlockSpec((_RB, 1), lambda i: (i, 0)),
        ],
        out_specs=pl.BlockSpec((_RB, D), lambda i: (i, 0)),
        out_shape=jax.ShapeDtypeStruct((N, D), jnp.float32),
    )(q0, q1, self2, b2, dis)


def kernel(x, edge_index, W1, b1, W2, b2):
    src = edge_index[0].astype(jnp.int32)
    dst = edge_index[1].astype(jnp.int32)
    b1r = b1.reshape(1, D)
    b2r = b2.reshape(1, D)
    zrows = jnp.zeros((RPT, D), jnp.float32)

    degp = _sc_degree(dst)
    deg = degp.reshape(NW, N).sum(0).reshape(N, 1) + 1.0

    xs1, self1, dis, inv = _tc_prep(x, W1, deg)

    a = _sc_aggregate(xs1, src, dst, zrows)
    xs2, self2 = _tc_mid(a[:N], a[N:], self1, b1r, W2, dis, inv)

    q = _sc_aggregate(xs2, src, dst, zrows)
    return _tc_final(q[:N], q[N:], self2, b2r, dis)


# speedup vs baseline: 1.0350x; 1.0350x over previous
import functools

import jax
import jax.numpy as jnp
from jax import lax
from jax.experimental import pallas as pl
from jax.experimental.pallas import tpu as pltpu
from jax.experimental.pallas import tpu_sc as plsc

N = 10000
D = 128
E = 320000

NC = 2
NS = 16
NW = NC * NS
LANES = 16

EPW = E // NW
K = 80
NCH = EPW // K
RPT = 624
ZR = 208

_mesh = plsc.VectorSubcoreMesh(core_axis_name="c", subcore_axis_name="s")


NBUF = 5
NIT = NCH // NBUF


@functools.partial(
    pl.kernel,
    mesh=_mesh,
    out_type=jax.ShapeDtypeStruct((NW * N,), jnp.float32),
    compiler_params=pltpu.CompilerParams(needs_layout_passes=False),
    scratch_types=[
        pltpu.VMEM((EPW,), jnp.int32),
        pltpu.VMEM((N,), jnp.float32),
    ],
)
def _sc_degree(dst_hbm, out_hbm, didx, cnt):
    c = lax.axis_index("c")
    s = lax.axis_index("s")
    wid = s * NC + c

    pltpu.sync_copy(dst_hbm.at[pl.ds(wid * EPW, EPW)], didx)

    zero16 = jnp.zeros((LANES,), jnp.float32)
    one16 = jnp.ones((LANES,), jnp.float32)

    def zfill(i, _):
        cnt[pl.ds(i * LANES, LANES)] = zero16
        return 0
    lax.fori_loop(0, N // LANES, zfill, 0)

    def count(j, _):
        idx = didx[pl.ds(j * LANES, LANES)]
        plsc.addupdate_scatter(cnt, [idx], one16)
        return 0
    lax.fori_loop(0, EPW // LANES, count, 0)

    pltpu.sync_copy(cnt, out_hbm.at[pl.ds(wid * N, N)])


KA = 40
NBA = 6
NCHA = EPW // KA
NITA = NCHA // NBA
TAIL = NCHA - NITA * NBA


@functools.partial(
    pl.kernel,
    mesh=_mesh,
    out_type=jax.ShapeDtypeStruct((NC * N, D), jnp.float32),
    scratch_types=[
        [pltpu.VMEM((KA,), jnp.int32)] * NBA,
        [pltpu.VMEM((KA,), jnp.int32)] * NBA,
        [pltpu.VMEM((KA, D), jnp.float32)] * NBA,
        pltpu.VMEM_SHARED((N, D), jnp.float32),
        [pltpu.SemaphoreType.DMA] * NBA,
        [pltpu.SemaphoreType.DMA] * NBA,
        [pltpu.SemaphoreType.DMA] * NBA,
    ],
)
def _sc_aggregate(xs_hbm, src_hbm, dst_hbm, zeros_hbm, out_hbm,
                  sidx, didx, rows, acc, isem, gsem, ssem):
    c = lax.axis_index("c")
    s = lax.axis_index("s")
    wid = s * NC + c

    rbase = s * RPT
    pltpu.sync_copy(zeros_hbm.at[pl.ds(0, RPT)],
                    acc.at[pl.ds(rbase, RPT)])

    @pl.when(s == NS - 1)
    def _ztail():
        pltpu.sync_copy(zeros_hbm.at[pl.ds(0, 16)], acc.at[pl.ds(N - 16, 16)])

    plsc.subcore_barrier()

    ebase = wid * EPW

    def icopy(i, b):
        eb = ebase + i * KA
        pltpu.async_copy(src_hbm.at[pl.ds(eb, KA)], sidx[b], isem[b])
        pltpu.async_copy(dst_hbm.at[pl.ds(eb, KA)], didx[b], isem[b])

    def iwait(b):
        pltpu.make_async_copy(src_hbm.at[pl.ds(0, KA)], sidx[b],
                              isem[b]).wait()
        pltpu.make_async_copy(dst_hbm.at[pl.ds(0, KA)], didx[b],
                              isem[b]).wait()

    def gather(b):
        pltpu.async_copy(xs_hbm.at[sidx[b]], rows[b], gsem[b])

    def gwait(b):
        pltpu.make_async_copy(xs_hbm.at[sidx[b]], rows[b], gsem[b]).wait()

    def scatter(b):
        pltpu.async_copy(rows[b], acc.at[didx[b]], ssem[b], add=True)

    def swait(b):
        pltpu.make_async_copy(rows[b], acc.at[didx[b]], ssem[b]).wait()

    for b in range(NBA):
        icopy(b, b)

    def step(g, _):
        for b in range(NBA):
            iwait(b)
            gather(b)
        for b in range(NBA):
            gwait(b)
            scatter(b)

        @pl.when(g < NITA - 1)
        def _prefetch():
            for b in range(NBA):
                swait(b)
                icopy(g * NBA + NBA + b, b)
        return 0
    lax.fori_loop(0, NITA, step, 0)

    for b in range(TAIL):
        swait(b)
        icopy(NITA * NBA + b, b)
    for b in range(TAIL):
        iwait(b)
        gather(b)
    for b in range(TAIL):
        gwait(b)
        scatter(b)
    for b in range(NBA):
        swait(b)

    plsc.subcore_barrier()
    pltpu.sync_copy(acc.at[pl.ds(rbase, RPT)],
                    out_hbm.at[pl.ds(c * N + rbase, RPT)])

    @pl.when(s == NS - 1)
    def _otail():
        pltpu.sync_copy(acc.at[pl.ds(N - 16, 16)],
                        out_hbm.at[pl.ds(c * N + N - 16, 16)])


_RB = 1000
_HIGH = lax.Precision.HIGHEST


def _prep_body(x_ref, w_ref, deg_ref, xs_ref, self_ref, dis_ref, inv_ref):
    deg = deg_ref[...]
    dis = lax.rsqrt(deg)
    inv = 1.0 / deg
    xw = jnp.dot(x_ref[...], w_ref[...], precision=_HIGH)
    xs_ref[...] = xw * dis
    self_ref[...] = xw * inv
    dis_ref[...] = dis
    inv_ref[...] = inv


def _tc_prep(x, w1, deg):
    grid = (N // _RB,)
    return pl.pallas_call(
        _prep_body,
        grid=grid,
        in_specs=[
            pl.BlockSpec((_RB, D), lambda i: (i, 0)),
            pl.BlockSpec((D, D), lambda i: (0, 0)),
            pl.BlockSpec((_RB, 1), lambda i: (i, 0)),
        ],
        out_specs=[
            pl.BlockSpec((_RB, D), lambda i: (i, 0)),
            pl.BlockSpec((_RB, D), lambda i: (i, 0)),
            pl.BlockSpec((_RB, 1), lambda i: (i, 0)),
            pl.BlockSpec((_RB, 1), lambda i: (i, 0)),
        ],
        out_shape=[
            jax.ShapeDtypeStruct((N, D), jnp.float32),
            jax.ShapeDtypeStruct((N, D), jnp.float32),
            jax.ShapeDtypeStruct((N, 1), jnp.float32),
            jax.ShapeDtypeStruct((N, 1), jnp.float32),
        ],
    )(x, w1, deg)


def _mid_body(q0_ref, q1_ref, self_ref, b_ref, w_ref, dis_ref, inv_ref,
              xs_ref, self2_ref):
    dis = dis_ref[...]
    inv = inv_ref[...]
    h = dis * (q0_ref[...] + q1_ref[...]) + self_ref[...] + b_ref[...]
    h = jnp.maximum(h, 0.0)
    xw = jnp.dot(h, w_ref[...], precision=_HIGH)
    xs_ref[...] = xw * dis
    self2_ref[...] = xw * inv


def _tc_mid(q0, q1, self1, b1, w2, dis, inv):
    grid = (N // _RB,)
    return pl.pallas_call(
        _mid_body,
        grid=grid,
        in_specs=[
            pl.BlockSpec((_RB, D), lambda i: (i, 0)),
            pl.BlockSpec((_RB, D), lambda i: (i, 0)),
            pl.BlockSpec((_RB, D), lambda i: (i, 0)),
            pl.BlockSpec((1, D), lambda i: (0, 0)),
            pl.BlockSpec((D, D), lambda i: (0, 0)),
            pl.BlockSpec((_RB, 1), lambda i: (i, 0)),
            pl.BlockSpec((_RB, 1), lambda i: (i, 0)),
        ],
        out_specs=[
            pl.BlockSpec((_RB, D), lambda i: (i, 0)),
            pl.BlockSpec((_RB, D), lambda i: (i, 0)),
        ],
        out_shape=[
            jax.ShapeDtypeStruct((N, D), jnp.float32),
            jax.ShapeDtypeStruct((N, D), jnp.float32),
        ],
    )(q0, q1, self1, b1, w2, dis, inv)


def _final_body(q0_ref, q1_ref, self_ref, b_ref, dis_ref, out_ref):
    out_ref[...] = (dis_ref[...] * (q0_ref[...] + q1_ref[...])
                    + self_ref[...] + b_ref[...])


def _tc_final(q0, q1, self2, b2, dis):
    grid = (N // _RB,)
    return pl.pallas_call(
        _final_body,
        grid=grid,
        in_specs=[
            pl.BlockSpec((_RB, D), lambda i: (i, 0)),
            pl.BlockSpec((_RB, D), lambda i: (i, 0)),
            pl.BlockSpec((_RB, D), lambda i: (i, 0)),
            pl.BlockSpec((1, D), lambda i: (0, 0)),
            pl.BlockSpec((_RB, 1), lambda i: (i, 0)),
        ],
        out_specs=pl.BlockSpec((_RB, D), lambda i: (i, 0)),
        out_shape=jax.ShapeDtypeStruct((N, D), jnp.float32),
    )(q0, q1, self2, b2, dis)


def kernel(x, edge_index, W1, b1, W2, b2):
    src = edge_index[0].astype(jnp.int32)
    dst = edge_index[1].astype(jnp.int32)
    b1r = b1.reshape(1, D)
    b2r = b2.reshape(1, D)
    zrows = jnp.zeros((RPT, D), jnp.float32)

    degp = _sc_degree(dst)
    deg = degp.reshape(NW, N).sum(0).reshape(N, 1) + 1.0

    xs1, self1, dis, inv = _tc_prep(x, W1, deg)

    a = _sc_aggregate(xs1, src, dst, zrows)
    xs2, self2 = _tc_mid(a[:N], a[N:], self1, b1r, W2, dis, inv)

    q = _sc_aggregate(xs2, src, dst, zrows)
    return _tc_final(q[:N], q[N:], self2, b2r, dis)
